# baseline (speedup 1.0000x reference)
import functools

import jax
import jax.numpy as jnp
from jax import lax
from jax.experimental import pallas as pl
from jax.experimental.pallas import tpu as pltpu
from jax.experimental.pallas import tpu_sc as plsc

_NUM_CORES = 2
_NUM_SUBCORES = 16
_NUM_WORKERS = _NUM_CORES * _NUM_SUBCORES
_LANES = 16

_DIM = 64
_UROWS = 128


@functools.cache
def _build_transpose(V):
  n_units = V // _UROWS
  rem = V - n_units * _UROWS
  iters = -(-n_units // _NUM_WORKERS)
  if iters % 2:
    iters += 1
  ubytes = _UROWS * _DIM
  mesh = plsc.VectorSubcoreMesh(core_axis_name="c", subcore_axis_name="s")

  @functools.partial(
      pl.kernel,
      out_type=jax.ShapeDtypeStruct((V * _DIM,), jnp.float32),
      mesh=mesh,
      scratch_types=[
          pltpu.VMEM((2, _DIM, _UROWS), jnp.float32),
          pltpu.VMEM((2, _UROWS * _DIM), jnp.float32),
          pltpu.VMEM((rem * _DIM,), jnp.float32) if rem else None,
          pltpu.SemaphoreType.DMA((2,)),
          pltpu.SemaphoreType.DMA((2,)),
      ],
      compiler_params=pltpu.CompilerParams(
          use_tc_tiling_on_sc=True, needs_layout_passes=False),
  )
  def k1(tabT_hbm, tail_hbm, lin_hbm, slab_v, obuf_v, tail_v, rsem, wsem):
    wid = lax.axis_index("s") * _NUM_CORES + lax.axis_index("c")
    iota = lax.iota(jnp.int32, _LANES)
    row_vecs = [iota + j0 for j0 in range(0, _DIM, _LANES)]

    def unit_of(u):
      return wid + u * _NUM_WORKERS

    def issue_read(u, par):
      c = unit_of(u)

      @pl.when(c < n_units)
      def _():
        pltpu.async_copy(
            tabT_hbm.at[:, pl.ds(c * _UROWS, _UROWS)],
            slab_v.at[par], rsem.at[par])

    def wait_read(u, par):
      @pl.when(unit_of(u) < n_units)
      def _():
        pltpu.make_async_copy(
            tabT_hbm.at[:, pl.ds(0, _UROWS)], slab_v.at[par], rsem.at[par]
        ).wait()

    def wait_write(u, par):
      @pl.when(unit_of(u) < n_units)
      def _():
        pltpu.make_async_copy(
            obuf_v.at[par], lin_hbm.at[pl.ds(0, ubytes)], wsem.at[par]
        ).wait()

    def transpose_and_write(u, par):
      c = unit_of(u)

      @pl.when(c < n_units)
      def _():
        def tb(b16, carry):
          for bs in range(_LANES):
            b = b16 * _LANES + bs
            colv = jnp.full((_LANES,), 0, jnp.int32) + b
            for g, rowv in enumerate(row_vecs):
              v = plsc.load_gather(slab_v.at[par], [rowv, colv])
              obuf_v[par, pl.ds(b * _DIM + g * _LANES, _LANES)] = v
          return carry

        lax.fori_loop(0, _UROWS // _LANES, tb, 0)
        pltpu.async_copy(
            obuf_v.at[par], lin_hbm.at[pl.ds(c * ubytes, ubytes)],
            wsem.at[par])

    issue_read(0, 0)

    def body(p, carry):
      for off, par in ((0, 0), (1, 1)):
        u = 2 * p + off
        issue_read(u + 1, 1 - par)
        wait_read(u, par)

        @pl.when(u >= 2)
        def _():
          wait_write(u - 2, par)

        transpose_and_write(u, par)
      return carry

    lax.fori_loop(0, iters // 2, body, 0)
    wait_write(iters - 2, 0)
    wait_write(iters - 1, 1)

    if rem:
      @pl.when(wid == 0)
      def _tail():
        pltpu.sync_copy(tail_hbm, tail_v)
        pltpu.sync_copy(tail_v, lin_hbm.at[pl.ds(n_units * ubytes, rem * _DIM)])

  return k1


_CHUNK = 256
_NBUF = 6
_LAG = 3


@functools.cache
def _build_gather(B, V):
  assert B % (_NUM_WORKERS * _CHUNK) == 0
  b_per_w = B // _NUM_WORKERS
  n_chunks = b_per_w // _CHUNK
  mesh = plsc.VectorSubcoreMesh(core_axis_name="c", subcore_axis_name="s")

  @functools.partial(
      pl.kernel,
      out_type=jax.ShapeDtypeStruct((B, _DIM), jnp.float32),
      mesh=mesh,
      scratch_types=[
          pltpu.VMEM((n_chunks, _CHUNK), jnp.int32),
          pltpu.VMEM((_NBUF, _CHUNK, _DIM), jnp.float32),
          pltpu.SemaphoreType.DMA((_NBUF,)),
          pltpu.SemaphoreType.DMA((_NBUF,)),
      ],
      compiler_params=pltpu.CompilerParams(use_tc_tiling_on_sc=False),
  )
  def k2(idx_hbm, table_hbm, out_hbm, idx_v, rows_v, gsem, wsem):
    wid = lax.axis_index("s") * _NUM_CORES + lax.axis_index("c")
    base = wid * b_per_w
    pltpu.sync_copy(idx_hbm.at[wid], idx_v)

    def step(c, carry):
      b = lax.rem(c, _NBUF)

      @pl.when(c < n_chunks)
      def _issue_gather():
        @pl.when(c >= _NBUF)
        def _():
          pltpu.make_async_copy(
              rows_v.at[b], out_hbm.at[pl.ds(base, _CHUNK)], wsem.at[b]
          ).wait()
        pltpu.async_copy(table_hbm.at[idx_v.at[c]], rows_v.at[b], gsem.at[b])

      @pl.when(c >= _LAG)
      def _retire():
        cc = c - _LAG
        bb = lax.rem(cc, _NBUF)
        pltpu.make_async_copy(
            table_hbm.at[idx_v.at[cc]], rows_v.at[bb], gsem.at[bb]
        ).wait()
        pltpu.async_copy(
            rows_v.at[bb], out_hbm.at[pl.ds(base + cc * _CHUNK, _CHUNK)],
            wsem.at[bb])

      return carry

    lax.fori_loop(0, n_chunks + _LAG, step, 0)

    for b in range(_NBUF):
      pltpu.make_async_copy(
          rows_v.at[b], out_hbm.at[pl.ds(base, _CHUNK)], wsem.at[b]
      ).wait()

  return k2


def kernel(token_ids, embeddings):
  n_rows, n_cols = token_ids.shape
  B = n_rows * n_cols
  V = embeddings.shape[0]
  n_units = V // _UROWS
  tail = embeddings[n_units * _UROWS:, :].reshape(-1)
  tab_lin = _build_transpose(V)(embeddings.T, tail).reshape(V, _DIM)
  idx = token_ids.astype(jnp.int32).reshape(
      _NUM_WORKERS, B // (_NUM_WORKERS * _CHUNK), _CHUNK)
  out = _build_gather(B, V)(idx, tab_lin)
  return out.reshape(n_rows, n_cols, _DIM)


# speedup vs baseline: 5.7976x; 5.7976x over previous
import functools

import jax
import jax.numpy as jnp
from jax import lax
from jax.experimental import pallas as pl
from jax.experimental.pallas import tpu as pltpu
from jax.experimental.pallas import tpu_sc as plsc

_NUM_CORES = 2
_NUM_SUBCORES = 16
_NUM_WORKERS = _NUM_CORES * _NUM_SUBCORES
_LANES = 16

_DIM = 64
_UROWS = 256


@functools.cache
def _build_transpose(V):
  n_units = V // _UROWS
  rem = V - n_units * _UROWS
  iters = -(-n_units // _NUM_WORKERS)
  if iters % 2:
    iters += 1
  ubytes = _UROWS * _DIM
  mesh = plsc.VectorSubcoreMesh(core_axis_name="c", subcore_axis_name="s")

  @functools.partial(
      pl.kernel,
      out_type=jax.ShapeDtypeStruct((V * _DIM,), jnp.float32),
      mesh=mesh,
      scratch_types=[
          pltpu.VMEM((2, _DIM, _UROWS), jnp.float32),
          pltpu.VMEM((2, _UROWS * _DIM), jnp.float32),
          pltpu.VMEM((rem * _DIM,), jnp.float32) if rem else None,
          pltpu.SemaphoreType.DMA((2,)),
          pltpu.SemaphoreType.DMA((2,)),
      ],
      compiler_params=pltpu.CompilerParams(
          use_tc_tiling_on_sc=True, needs_layout_passes=False),
  )
  def k1(tabT_hbm, tail_hbm, lin_hbm, slab_v, obuf_v, tail_v, rsem, wsem):
    wid = lax.axis_index("s") * _NUM_CORES + lax.axis_index("c")
    iota = lax.iota(jnp.int32, _LANES)

    def unit_of(u):
      return wid + u * _NUM_WORKERS

    def issue_read(u, par):
      c = unit_of(u)

      @pl.when(c < n_units)
      def _():
        pltpu.async_copy(
            tabT_hbm.at[:, pl.ds(c * _UROWS, _UROWS)],
            slab_v.at[par], rsem.at[par])

    def wait_read(u, par):
      @pl.when(unit_of(u) < n_units)
      def _():
        pltpu.make_async_copy(
            tabT_hbm.at[:, pl.ds(0, _UROWS)],
            slab_v.at[par], rsem.at[par]
        ).wait()

    def wait_write(u, par):
      @pl.when(unit_of(u) < n_units)
      def _():
        pltpu.make_async_copy(
            obuf_v.at[par], lin_hbm.at[pl.ds(0, ubytes)], wsem.at[par]
        ).wait()

    def transpose_and_write(u, par):
      c = unit_of(u)

      @pl.when(c < n_units)
      def _():
        parv = jnp.full((_LANES,), par, jnp.int32)

        @plsc.parallel_loop(0, _UROWS * (_DIM // _LANES), unroll=8)
        def _tq(q):
          rowv = iota + ((q & 3) << 4)
          colv = (iota + (q >> 2)) & (_UROWS - 1)
          v = plsc.load_gather(slab_v.at[par], [rowv, colv])
          plsc.store_scatter(obuf_v, [parv, (colv << 6) + rowv], v)
        pltpu.async_copy(
            obuf_v.at[par], lin_hbm.at[pl.ds(c * ubytes, ubytes)],
            wsem.at[par])

    issue_read(0, 0)

    def body(p, carry):
      for off, par in ((0, 0), (1, 1)):
        u = 2 * p + off
        issue_read(u + 1, 1 - par)
        wait_read(u, par)

        @pl.when(u >= 2)
        def _():
          wait_write(u - 2, par)

        transpose_and_write(u, par)
      return carry

    lax.fori_loop(0, iters // 2, body, 0)
    wait_write(iters - 2, 0)
    wait_write(iters - 1, 1)

    if rem:
      @pl.when(wid == 0)
      def _tail():
        pltpu.sync_copy(tail_hbm, tail_v)
        pltpu.sync_copy(tail_v, lin_hbm.at[pl.ds(n_units * ubytes, rem * _DIM)])

  return k1


_GCHUNK = 128
_GNBUF = 4
_GLAG = 2


@functools.cache
def _build_gather(n_rows, n_cols, V):
  nbc = n_rows // _GCHUNK
  n_slabs = n_cols * nbc
  per_w = n_slabs // _NUM_WORKERS
  assert per_w * _NUM_WORKERS == n_slabs
  mesh = plsc.VectorSubcoreMesh(core_axis_name="c", subcore_axis_name="s")

  @functools.partial(
      pl.kernel,
      out_type=jax.ShapeDtypeStruct(
          (n_cols, _DIM // 8, nbc, 8, _GCHUNK), jnp.float32),
      mesh=mesh,
      scratch_types=[
          pltpu.VMEM((per_w, _GCHUNK), jnp.int32),
          pltpu.VMEM((_GNBUF, _GCHUNK, _DIM), jnp.float32),
          pltpu.VMEM((_GNBUF, _DIM // 8, 8, _GCHUNK), jnp.float32),
          pltpu.SemaphoreType.DMA((_GNBUF,)),
          pltpu.SemaphoreType.DMA((_GNBUF,)),
      ],
      compiler_params=pltpu.CompilerParams(
          use_tc_tiling_on_sc=False, needs_layout_passes=False),
  )
  def k2(idx_hbm, table_hbm, out_hbm, idx_v, rows_v, tbuf_v, gsem, wsem):
    wid = lax.axis_index("s") * _NUM_CORES + lax.axis_index("c")
    s_base = wid * per_w
    iota = lax.iota(jnp.int32, _LANES)
    pltpu.sync_copy(idx_hbm.at[wid], idx_v)

    def out_slice(kk):
      s = s_base + kk
      return out_hbm.at[s // nbc, :, lax.rem(s, nbc)]

    def step(c, carry):
      b = lax.rem(c, _GNBUF)

      @pl.when(c < per_w)
      def _issue_gather():
        @pl.when(c >= _GNBUF)
        def _():
          pltpu.make_async_copy(
              tbuf_v.at[b], out_slice(0), wsem.at[b]).wait()
        pltpu.async_copy(table_hbm.at[idx_v.at[c]], rows_v.at[b], gsem.at[b])

      @pl.when(c >= _GLAG)
      def _retire():
        cc = c - _GLAG
        bb = lax.rem(cc, _GNBUF)
        pltpu.make_async_copy(
            table_hbm.at[idx_v.at[cc]], rows_v.at[bb], gsem.at[bb]
        ).wait()
        bvec = jnp.full((_LANES,), 0, jnp.int32) + bb

        @plsc.parallel_loop(0, _GCHUNK * (_DIM // _LANES), unroll=8)
        def _tq(q):
          rowv = iota + ((q & 3) << 4)
          colv = (iota + (q >> 2)) & (_GCHUNK - 1)
          v = plsc.load_gather(rows_v, [bvec, colv, rowv])
          plsc.store_scatter(
              tbuf_v, [bvec, rowv >> 3, rowv & 7, colv], v)

        pltpu.async_copy(tbuf_v.at[bb], out_slice(cc), wsem.at[bb])

      return carry

    lax.fori_loop(0, per_w + _GLAG, step, 0)

    for b in range(_GNBUF):
      pltpu.make_async_copy(tbuf_v.at[b], out_slice(0), wsem.at[b]).wait()

  return k2


def kernel(token_ids, embeddings):
  n_rows, n_cols = token_ids.shape
  V = embeddings.shape[0]
  n_units = V // _UROWS
  tail = embeddings[n_units * _UROWS:, :].reshape(-1)
  tab_lin = _build_transpose(V)(embeddings.T, tail).reshape(V, _DIM)
  idx = token_ids.astype(jnp.int32).T.reshape(
      _NUM_WORKERS, (n_cols * n_rows) // (_NUM_WORKERS * _GCHUNK), _GCHUNK)
  out5 = _build_gather(n_rows, n_cols, V)(idx, tab_lin)
  return out5.transpose(2, 4, 0, 1, 3).reshape(n_rows, n_cols, _DIM)
